# baseline (speedup 1.0000x reference)
import functools

import jax
import jax.numpy as jnp
from jax.experimental import pallas as pl
from jax.experimental.pallas import tpu as pltpu

_G = 128
_BN = 1024


def _leaky(x):
    return jnp.where(x >= 0, x, 0.01 * x)


def _gate_stats_kernel(h_ref, bcol_ref, W1_ref, b1_ref, W2_ref, b2_ref,
                       W3_ref, b3_ref, g_ref, m_out_ref, s_out_ref,
                       m_ref, s_ref, *, nb, G):
    i = pl.program_id(0)

    @pl.when(i == 0)
    def _():
        m_ref[...] = jnp.full_like(m_ref, -1e30)
        s_ref[...] = jnp.zeros_like(s_ref)

    x = _leaky(jnp.dot(h_ref[...], W1_ref[...],
                       preferred_element_type=jnp.float32) + b1_ref[...])
    x = _leaky(jnp.dot(x.astype(jnp.bfloat16), W2_ref[...],
                       preferred_element_type=jnp.float32) + b2_ref[...])
    g = jnp.dot(x, W3_ref[...],
                preferred_element_type=jnp.float32) + b3_ref[...]
    g_ref[...] = g

    ids = bcol_ref[...]
    seg = jax.lax.broadcasted_iota(jnp.int32, (1, G), 1)
    hit = ids == seg
    maskf = hit.astype(jnp.float32)

    m_old = m_ref[...]
    m_blk = jnp.max(jnp.where(hit, g, -1e30), axis=0, keepdims=True)
    m_new = jnp.maximum(m_old, m_blk)
    scale = jnp.exp(m_old - m_new)
    m_gather = jnp.sum(maskf * m_new, axis=1, keepdims=True)
    e = jnp.exp(g - m_gather)
    s_blk = jnp.sum(maskf * e, axis=0, keepdims=True)
    s_ref[...] = s_ref[...] * scale + s_blk
    m_ref[...] = m_new

    @pl.when(i == nb - 1)
    def _():
        m_out_ref[...] = m_ref[...]
        s_out_ref[...] = s_ref[...]


def _pool_kernel(h_ref, bcol_ref, brow_ref, g_ref, m_ref, s_ref,
                 scores_ref, pool_ref, *, G):
    i = pl.program_id(0)
    ids = bcol_ref[...]
    seg = jax.lax.broadcasted_iota(jnp.int32, (1, G), 1)
    maskf = (ids == seg).astype(jnp.float32)
    m = m_ref[...]
    s = s_ref[...]
    m_gather = jnp.sum(maskf * m, axis=1, keepdims=True)
    s_gather = jnp.sum(maskf * s, axis=1, keepdims=True)
    e = jnp.exp(g_ref[...] - m_gather)
    sc = e / jnp.where(s_gather > 0, s_gather, 1.0)
    scores_ref[...] = sc

    hs = (h_ref[...] * sc.astype(jnp.bfloat16)).astype(jnp.bfloat16)
    ids_row = brow_ref[0]
    seg_col = jax.lax.broadcasted_iota(jnp.int32, (G, 1), 0)
    maskT = (seg_col == ids_row).astype(jnp.bfloat16)
    part = jnp.dot(maskT, hs, preferred_element_type=jnp.float32)

    @pl.when(i == 0)
    def _():
        pool_ref[...] = jnp.zeros_like(pool_ref)

    pool_ref[...] += part


def kernel(h, batch, W1, b1, W2, b2, W3, b3):
    N, D = h.shape
    H = W1.shape[1]
    G = _G
    BN = _BN
    nb = -(-N // BN)
    npad = nb * BN

    hp = jnp.pad(h, ((0, npad - N), (0, 0))).astype(jnp.bfloat16)
    bp = jnp.pad(batch, (0, npad - N), constant_values=G)
    bcol = bp.reshape(npad, 1)
    brow = bp.reshape(nb, 1, BN)
    b1r = b1.reshape(1, H)
    b2r = b2.reshape(1, H)
    b3r = b3.reshape(1, 1)

    g, m, s = pl.pallas_call(
        functools.partial(_gate_stats_kernel, nb=nb, G=G),
        grid=(nb,),
        in_specs=[
            pl.BlockSpec((BN, D), lambda i: (i, 0)),
            pl.BlockSpec((BN, 1), lambda i: (i, 0)),
            pl.BlockSpec((D, H), lambda i: (0, 0)),
            pl.BlockSpec((1, H), lambda i: (0, 0)),
            pl.BlockSpec((H, H), lambda i: (0, 0)),
            pl.BlockSpec((1, H), lambda i: (0, 0)),
            pl.BlockSpec((H, 1), lambda i: (0, 0)),
            pl.BlockSpec((1, 1), lambda i: (0, 0)),
        ],
        out_specs=[
            pl.BlockSpec((BN, 1), lambda i: (i, 0)),
            pl.BlockSpec((1, G), lambda i: (0, 0)),
            pl.BlockSpec((1, G), lambda i: (0, 0)),
        ],
        out_shape=[
            jax.ShapeDtypeStruct((npad, 1), jnp.float32),
            jax.ShapeDtypeStruct((1, G), jnp.float32),
            jax.ShapeDtypeStruct((1, G), jnp.float32),
        ],
        scratch_shapes=[
            pltpu.VMEM((1, G), jnp.float32),
            pltpu.VMEM((1, G), jnp.float32),
        ],
    )(hp, bcol, W1.astype(jnp.bfloat16), b1r, W2.astype(jnp.bfloat16), b2r,
      W3, b3r)

    scores, pool = pl.pallas_call(
        functools.partial(_pool_kernel, G=G),
        grid=(nb,),
        in_specs=[
            pl.BlockSpec((BN, D), lambda i: (i, 0)),
            pl.BlockSpec((BN, 1), lambda i: (i, 0)),
            pl.BlockSpec((1, 1, BN), lambda i: (i, 0, 0)),
            pl.BlockSpec((BN, 1), lambda i: (i, 0)),
            pl.BlockSpec((1, G), lambda i: (0, 0)),
            pl.BlockSpec((1, G), lambda i: (0, 0)),
        ],
        out_specs=[
            pl.BlockSpec((BN, 1), lambda i: (i, 0)),
            pl.BlockSpec((G, D), lambda i: (0, 0)),
        ],
        out_shape=[
            jax.ShapeDtypeStruct((npad, 1), jnp.float32),
            jax.ShapeDtypeStruct((G, D), jnp.float32),
        ],
    )(hp, bcol, brow, g, m, s)

    return (pool, scores[:N])


# speedup vs baseline: 1.1708x; 1.1708x over previous
import functools

import jax
import jax.numpy as jnp
from jax.experimental import pallas as pl
from jax.experimental.pallas import tpu as pltpu

_G = 128
_BN = 2048


def _leaky(x):
    return jnp.where(x >= 0, x, 0.01 * x)


def _gate_stats_kernel(h_ref, bcol_ref, W1_ref, b1_ref, W2_ref, b2_ref,
                       W3_ref, b3_ref, g_ref, m_out_ref, s_out_ref,
                       m_ref, s_ref, *, nb, G):
    i = pl.program_id(0)

    @pl.when(i == 0)
    def _():
        m_ref[...] = jnp.full_like(m_ref, -1e30)
        s_ref[...] = jnp.zeros_like(s_ref)

    x = _leaky(jnp.dot(h_ref[...], W1_ref[...],
                       preferred_element_type=jnp.float32) + b1_ref[...])
    x = _leaky(jnp.dot(x.astype(jnp.bfloat16), W2_ref[...],
                       preferred_element_type=jnp.float32) + b2_ref[...])
    g = jnp.dot(x, W3_ref[...],
                preferred_element_type=jnp.float32) + b3_ref[...]
    g_ref[...] = g

    ids = bcol_ref[...]
    seg = jax.lax.broadcasted_iota(jnp.int32, (1, G), 1)
    hit = ids == seg
    maskf = hit.astype(jnp.float32)

    m_old = m_ref[...]
    m_blk = jnp.max(jnp.where(hit, g, -1e30), axis=0, keepdims=True)
    m_new = jnp.maximum(m_old, m_blk)
    scale = jnp.exp(m_old - m_new)
    m_gather = jnp.sum(maskf * m_new, axis=1, keepdims=True)
    e = jnp.exp(g - m_gather)
    s_blk = jnp.sum(maskf * e, axis=0, keepdims=True)
    s_ref[...] = s_ref[...] * scale + s_blk
    m_ref[...] = m_new

    @pl.when(i == nb - 1)
    def _():
        m_out_ref[...] = m_ref[...]
        s_out_ref[...] = s_ref[...]


def _pool_kernel(h_ref, bcol_ref, brow_ref, g_ref, m_ref, s_ref,
                 scores_ref, pool_ref, *, G):
    i = pl.program_id(0)
    ids = bcol_ref[...]
    seg = jax.lax.broadcasted_iota(jnp.int32, (1, G), 1)
    maskf = (ids == seg).astype(jnp.float32)
    m = m_ref[...]
    s = s_ref[...]
    m_gather = jnp.sum(maskf * m, axis=1, keepdims=True)
    s_gather = jnp.sum(maskf * s, axis=1, keepdims=True)
    e = jnp.exp(g_ref[...] - m_gather)
    sc = e / jnp.where(s_gather > 0, s_gather, 1.0)
    scores_ref[...] = sc

    hs = (h_ref[...] * sc.astype(jnp.bfloat16)).astype(jnp.bfloat16)
    ids_row = brow_ref[0]
    seg_col = jax.lax.broadcasted_iota(jnp.int32, (G, 1), 0)
    maskT = (seg_col == ids_row).astype(jnp.bfloat16)
    part = jnp.dot(maskT, hs, preferred_element_type=jnp.float32)

    @pl.when(i == 0)
    def _():
        pool_ref[...] = jnp.zeros_like(pool_ref)

    pool_ref[...] += part


def kernel(h, batch, W1, b1, W2, b2, W3, b3):
    N, D = h.shape
    H = W1.shape[1]
    G = _G
    BN = _BN
    nb = -(-N // BN)
    npad = nb * BN

    hp = jnp.pad(h, ((0, npad - N), (0, 0))).astype(jnp.bfloat16)
    bp = jnp.pad(batch, (0, npad - N), constant_values=G)
    bcol = bp.reshape(npad, 1)
    brow = bp.reshape(nb, 1, BN)
    b1r = b1.reshape(1, H)
    b2r = b2.reshape(1, H)
    b3r = b3.reshape(1, 1)

    g, m, s = pl.pallas_call(
        functools.partial(_gate_stats_kernel, nb=nb, G=G),
        grid=(nb,),
        in_specs=[
            pl.BlockSpec((BN, D), lambda i: (i, 0)),
            pl.BlockSpec((BN, 1), lambda i: (i, 0)),
            pl.BlockSpec((D, H), lambda i: (0, 0)),
            pl.BlockSpec((1, H), lambda i: (0, 0)),
            pl.BlockSpec((H, H), lambda i: (0, 0)),
            pl.BlockSpec((1, H), lambda i: (0, 0)),
            pl.BlockSpec((H, 1), lambda i: (0, 0)),
            pl.BlockSpec((1, 1), lambda i: (0, 0)),
        ],
        out_specs=[
            pl.BlockSpec((BN, 1), lambda i: (i, 0)),
            pl.BlockSpec((1, G), lambda i: (0, 0)),
            pl.BlockSpec((1, G), lambda i: (0, 0)),
        ],
        out_shape=[
            jax.ShapeDtypeStruct((npad, 1), jnp.float32),
            jax.ShapeDtypeStruct((1, G), jnp.float32),
            jax.ShapeDtypeStruct((1, G), jnp.float32),
        ],
        scratch_shapes=[
            pltpu.VMEM((1, G), jnp.float32),
            pltpu.VMEM((1, G), jnp.float32),
        ],
    )(hp, bcol, W1.astype(jnp.bfloat16), b1r, W2.astype(jnp.bfloat16), b2r,
      W3, b3r)

    scores, pool = pl.pallas_call(
        functools.partial(_pool_kernel, G=G),
        grid=(nb,),
        in_specs=[
            pl.BlockSpec((BN, D), lambda i: (i, 0)),
            pl.BlockSpec((BN, 1), lambda i: (i, 0)),
            pl.BlockSpec((1, 1, BN), lambda i: (i, 0, 0)),
            pl.BlockSpec((BN, 1), lambda i: (i, 0)),
            pl.BlockSpec((1, G), lambda i: (0, 0)),
            pl.BlockSpec((1, G), lambda i: (0, 0)),
        ],
        out_specs=[
            pl.BlockSpec((BN, 1), lambda i: (i, 0)),
            pl.BlockSpec((G, D), lambda i: (0, 0)),
        ],
        out_shape=[
            jax.ShapeDtypeStruct((npad, 1), jnp.float32),
            jax.ShapeDtypeStruct((G, D), jnp.float32),
        ],
    )(hp, bcol, brow, g, m, s)

    return (pool, scores[:N])


# speedup vs baseline: 1.2370x; 1.0565x over previous
import functools

import jax
import jax.numpy as jnp
from jax.experimental import pallas as pl
from jax.experimental.pallas import tpu as pltpu

_G = 128
_BN = 4096


def _leaky(x):
    return jnp.where(x >= 0, x, 0.01 * x)


def _gate_stats_kernel(h_ref, bcol_ref, W1_ref, b1_ref, W2_ref, b2_ref,
                       W3_ref, b3_ref, g_ref, m_out_ref, s_out_ref,
                       m_ref, s_ref, *, nb, G):
    i = pl.program_id(0)

    @pl.when(i == 0)
    def _():
        m_ref[...] = jnp.full_like(m_ref, -1e30)
        s_ref[...] = jnp.zeros_like(s_ref)

    x = _leaky(jnp.dot(h_ref[...], W1_ref[...],
                       preferred_element_type=jnp.float32) + b1_ref[...])
    x = _leaky(jnp.dot(x.astype(jnp.bfloat16), W2_ref[...],
                       preferred_element_type=jnp.float32) + b2_ref[...])
    g = jnp.dot(x, W3_ref[...],
                preferred_element_type=jnp.float32) + b3_ref[...]
    g_ref[...] = g

    ids = bcol_ref[...]
    seg = jax.lax.broadcasted_iota(jnp.int32, (1, G), 1)
    hit = ids == seg
    maskf = hit.astype(jnp.float32)

    m_old = m_ref[...]
    m_blk = jnp.max(jnp.where(hit, g, -1e30), axis=0, keepdims=True)
    m_new = jnp.maximum(m_old, m_blk)
    scale = jnp.exp(m_old - m_new)
    m_gather = jnp.sum(maskf * m_new, axis=1, keepdims=True)
    e = jnp.exp(g - m_gather)
    s_blk = jnp.sum(maskf * e, axis=0, keepdims=True)
    s_ref[...] = s_ref[...] * scale + s_blk
    m_ref[...] = m_new

    @pl.when(i == nb - 1)
    def _():
        m_out_ref[...] = m_ref[...]
        s_out_ref[...] = s_ref[...]


def _pool_kernel(h_ref, bcol_ref, brow_ref, g_ref, m_ref, s_ref,
                 scores_ref, pool_ref, *, G):
    i = pl.program_id(0)
    ids = bcol_ref[...]
    seg = jax.lax.broadcasted_iota(jnp.int32, (1, G), 1)
    maskf = (ids == seg).astype(jnp.float32)
    m = m_ref[...]
    s = s_ref[...]
    m_gather = jnp.sum(maskf * m, axis=1, keepdims=True)
    s_gather = jnp.sum(maskf * s, axis=1, keepdims=True)
    e = jnp.exp(g_ref[...] - m_gather)
    sc = e / jnp.where(s_gather > 0, s_gather, 1.0)
    scores_ref[...] = sc

    hs = (h_ref[...] * sc.astype(jnp.bfloat16)).astype(jnp.bfloat16)
    ids_row = brow_ref[0]
    seg_col = jax.lax.broadcasted_iota(jnp.int32, (G, 1), 0)
    maskT = (seg_col == ids_row).astype(jnp.bfloat16)
    part = jnp.dot(maskT, hs, preferred_element_type=jnp.float32)

    @pl.when(i == 0)
    def _():
        pool_ref[...] = jnp.zeros_like(pool_ref)

    pool_ref[...] += part


def kernel(h, batch, W1, b1, W2, b2, W3, b3):
    N, D = h.shape
    H = W1.shape[1]
    G = _G
    BN = _BN
    nb = -(-N // BN)
    npad = nb * BN

    hp = jnp.pad(h, ((0, npad - N), (0, 0))).astype(jnp.bfloat16)
    bp = jnp.pad(batch, (0, npad - N), constant_values=G)
    bcol = bp.reshape(npad, 1)
    brow = bp.reshape(nb, 1, BN)
    b1r = b1.reshape(1, H)
    b2r = b2.reshape(1, H)
    b3r = b3.reshape(1, 1)

    g, m, s = pl.pallas_call(
        functools.partial(_gate_stats_kernel, nb=nb, G=G),
        grid=(nb,),
        in_specs=[
            pl.BlockSpec((BN, D), lambda i: (i, 0)),
            pl.BlockSpec((BN, 1), lambda i: (i, 0)),
            pl.BlockSpec((D, H), lambda i: (0, 0)),
            pl.BlockSpec((1, H), lambda i: (0, 0)),
            pl.BlockSpec((H, H), lambda i: (0, 0)),
            pl.BlockSpec((1, H), lambda i: (0, 0)),
            pl.BlockSpec((H, 1), lambda i: (0, 0)),
            pl.BlockSpec((1, 1), lambda i: (0, 0)),
        ],
        out_specs=[
            pl.BlockSpec((BN, 1), lambda i: (i, 0)),
            pl.BlockSpec((1, G), lambda i: (0, 0)),
            pl.BlockSpec((1, G), lambda i: (0, 0)),
        ],
        out_shape=[
            jax.ShapeDtypeStruct((npad, 1), jnp.float32),
            jax.ShapeDtypeStruct((1, G), jnp.float32),
            jax.ShapeDtypeStruct((1, G), jnp.float32),
        ],
        scratch_shapes=[
            pltpu.VMEM((1, G), jnp.float32),
            pltpu.VMEM((1, G), jnp.float32),
        ],
    )(hp, bcol, W1.astype(jnp.bfloat16), b1r, W2.astype(jnp.bfloat16), b2r,
      W3, b3r)

    scores, pool = pl.pallas_call(
        functools.partial(_pool_kernel, G=G),
        grid=(nb,),
        in_specs=[
            pl.BlockSpec((BN, D), lambda i: (i, 0)),
            pl.BlockSpec((BN, 1), lambda i: (i, 0)),
            pl.BlockSpec((1, 1, BN), lambda i: (i, 0, 0)),
            pl.BlockSpec((BN, 1), lambda i: (i, 0)),
            pl.BlockSpec((1, G), lambda i: (0, 0)),
            pl.BlockSpec((1, G), lambda i: (0, 0)),
        ],
        out_specs=[
            pl.BlockSpec((BN, 1), lambda i: (i, 0)),
            pl.BlockSpec((G, D), lambda i: (0, 0)),
        ],
        out_shape=[
            jax.ShapeDtypeStruct((npad, 1), jnp.float32),
            jax.ShapeDtypeStruct((G, D), jnp.float32),
        ],
    )(hp, bcol, brow, g, m, s)

    return (pool, scores[:N])


# speedup vs baseline: 1.7414x; 1.4078x over previous
import functools

import jax
import jax.numpy as jnp
from jax.experimental import pallas as pl
from jax.experimental.pallas import tpu as pltpu

_G = 128
_BN = 4096


def _leaky(x):
    return jnp.where(x >= 0, x, 0.01 * x)


def _valid_rows(i, bn, n):
    row = i * bn + jax.lax.broadcasted_iota(jnp.int32, (bn, 1), 0)
    return row < n


def _gate_stats_kernel(h_ref, bcol_ref, W1_ref, b1_ref, W2_ref, b2_ref,
                       W3_ref, b3_ref, g_ref, m_out_ref, s_out_ref,
                       m_ref, s_ref, *, nb, G, n):
    i = pl.program_id(0)

    @pl.when(i == 0)
    def _():
        m_ref[...] = jnp.full_like(m_ref, -1e30)
        s_ref[...] = jnp.zeros_like(s_ref)

    bn = h_ref.shape[0]
    hb = jnp.where(_valid_rows(i, bn, n), h_ref[...], 0.0)
    x = _leaky(jnp.dot(hb, W1_ref[...],
                       preferred_element_type=jnp.float32) + b1_ref[...])
    x = _leaky(jnp.dot(x, W2_ref[...],
                       preferred_element_type=jnp.float32) + b2_ref[...])
    g = jnp.dot(x, W3_ref[...],
                preferred_element_type=jnp.float32) + b3_ref[...]
    g_ref[...] = g

    ids = bcol_ref[...]
    seg = jax.lax.broadcasted_iota(jnp.int32, (1, G), 1)
    hit = ids == seg
    maskf = hit.astype(jnp.float32)

    m_old = m_ref[...]
    m_blk = jnp.max(jnp.where(hit, g, -1e30), axis=0, keepdims=True)
    m_new = jnp.maximum(m_old, m_blk)
    scale = jnp.exp(m_old - m_new)
    m_gather = jnp.sum(maskf * m_new, axis=1, keepdims=True)
    e = jnp.exp(g - m_gather)
    s_blk = jnp.sum(maskf * e, axis=0, keepdims=True)
    s_ref[...] = s_ref[...] * scale + s_blk
    m_ref[...] = m_new

    @pl.when(i == nb - 1)
    def _():
        m_out_ref[...] = m_ref[...]
        s_out_ref[...] = s_ref[...]


def _pool_kernel(h_ref, bcol_ref, brow_ref, g_ref, m_ref, s_ref,
                 scores_ref, pool_ref, *, G, n):
    i = pl.program_id(0)
    ids = bcol_ref[...]
    seg = jax.lax.broadcasted_iota(jnp.int32, (1, G), 1)
    maskf = (ids == seg).astype(jnp.float32)
    m = m_ref[...]
    s = s_ref[...]
    m_gather = jnp.sum(maskf * m, axis=1, keepdims=True)
    s_gather = jnp.sum(maskf * s, axis=1, keepdims=True)
    e = jnp.exp(g_ref[...] - m_gather)
    sc = e / jnp.where(s_gather > 0, s_gather, 1.0)
    scores_ref[...] = sc

    bn = h_ref.shape[0]
    hb = jnp.where(_valid_rows(i, bn, n), h_ref[...], 0.0)
    hs = hb * sc
    ids_row = brow_ref[0]
    seg_col = jax.lax.broadcasted_iota(jnp.int32, (G, 1), 0)
    maskT = (seg_col == ids_row).astype(jnp.float32)
    part = jnp.dot(maskT, hs, preferred_element_type=jnp.float32)

    @pl.when(i == 0)
    def _():
        pool_ref[...] = jnp.zeros_like(pool_ref)

    pool_ref[...] += part


def kernel(h, batch, W1, b1, W2, b2, W3, b3):
    N, D = h.shape
    H = W1.shape[1]
    G = _G
    BN = _BN
    nb = -(-N // BN)
    npad = nb * BN

    bp = jnp.pad(batch, (0, npad - N), constant_values=G)
    bcol = bp.reshape(npad, 1)
    brow = bp.reshape(nb, 1, BN)
    b1r = b1.reshape(1, H)
    b2r = b2.reshape(1, H)
    b3r = b3.reshape(1, 1)

    g, m, s = pl.pallas_call(
        functools.partial(_gate_stats_kernel, nb=nb, G=G, n=N),
        grid=(nb,),
        in_specs=[
            pl.BlockSpec((BN, D), lambda i: (i, 0)),
            pl.BlockSpec((BN, 1), lambda i: (i, 0)),
            pl.BlockSpec((D, H), lambda i: (0, 0)),
            pl.BlockSpec((1, H), lambda i: (0, 0)),
            pl.BlockSpec((H, H), lambda i: (0, 0)),
            pl.BlockSpec((1, H), lambda i: (0, 0)),
            pl.BlockSpec((H, 1), lambda i: (0, 0)),
            pl.BlockSpec((1, 1), lambda i: (0, 0)),
        ],
        out_specs=[
            pl.BlockSpec((BN, 1), lambda i: (i, 0)),
            pl.BlockSpec((1, G), lambda i: (0, 0)),
            pl.BlockSpec((1, G), lambda i: (0, 0)),
        ],
        out_shape=[
            jax.ShapeDtypeStruct((npad, 1), jnp.float32),
            jax.ShapeDtypeStruct((1, G), jnp.float32),
            jax.ShapeDtypeStruct((1, G), jnp.float32),
        ],
        scratch_shapes=[
            pltpu.VMEM((1, G), jnp.float32),
            pltpu.VMEM((1, G), jnp.float32),
        ],
    )(h, bcol, W1, b1r, W2, b2r, W3, b3r)

    scores, pool = pl.pallas_call(
        functools.partial(_pool_kernel, G=G, n=N),
        grid=(nb,),
        in_specs=[
            pl.BlockSpec((BN, D), lambda i: (i, 0)),
            pl.BlockSpec((BN, 1), lambda i: (i, 0)),
            pl.BlockSpec((1, 1, BN), lambda i: (i, 0, 0)),
            pl.BlockSpec((BN, 1), lambda i: (i, 0)),
            pl.BlockSpec((1, G), lambda i: (0, 0)),
            pl.BlockSpec((1, G), lambda i: (0, 0)),
        ],
        out_specs=[
            pl.BlockSpec((BN, 1), lambda i: (i, 0)),
            pl.BlockSpec((G, D), lambda i: (0, 0)),
        ],
        out_shape=[
            jax.ShapeDtypeStruct((N, 1), jnp.float32),
            jax.ShapeDtypeStruct((G, D), jnp.float32),
        ],
    )(h, bcol, brow, g, m, s)

    return (pool, scores)
